# baseline (speedup 1.0000x reference)
import jax
import jax.numpy as jnp
from jax.experimental import pallas as pl
from jax.experimental.pallas import tpu as pltpu

N_NODES = 32
LANES = 128
ROWS_TOTAL = 8192
G = 1024
GS = 4
K = N_NODES * GS


def _body(xi_t_ref, s_nd_ref, x_ref, o_ref):
    xi_t = xi_t_ref[...]
    s_nd = s_nd_ref[...]
    for s in range(G // GS):
        x = x_ref[s * GS : (s + 1) * GS, :]
        d = jnp.reshape(x[:, None, :] - xi_t[None, :, :], (K, LANES))
        c = 1.0 / d
        c = jnp.minimum(c, 1e24)
        cb = c.astype(jnp.bfloat16)
        nd = jnp.dot(s_nd, cb, preferred_element_type=jnp.float32)
        out = nd[0:GS, :] / nd[GS : 2 * GS, :]
        o_ref[s * GS : (s + 1) * GS, :] = jnp.where(
            jnp.isfinite(out), out, x * x
        )


def kernel(x, xi, fi, wi):
    f32 = jnp.float32
    xq = x.reshape(ROWS_TOTAL, LANES)
    fw = fi * wi
    eye = jnp.eye(GS, dtype=f32)
    s_nd = jnp.concatenate(
        [jnp.kron(eye, fw[None, :]), jnp.kron(eye, wi[None, :])], axis=0
    ).astype(jnp.bfloat16)
    xi_t = jnp.broadcast_to(xi[:, None], (N_NODES, LANES))

    grid = (ROWS_TOTAL // G,)
    out = pl.pallas_call(
        _body,
        grid=grid,
        in_specs=[
            pl.BlockSpec((N_NODES, LANES), lambda i: (0, 0)),
            pl.BlockSpec((2 * GS, K), lambda i: (0, 0)),
            pl.BlockSpec((G, LANES), lambda i: (i, 0)),
        ],
        out_specs=pl.BlockSpec((G, LANES), lambda i: (i, 0)),
        out_shape=jax.ShapeDtypeStruct((ROWS_TOTAL, LANES), f32),
        compiler_params=pltpu.CompilerParams(
            dimension_semantics=("arbitrary",),
        ),
    )(xi_t, s_nd, xq)
    return out.reshape(-1)


# speedup vs baseline: 1.0369x; 1.0369x over previous
import jax
import jax.numpy as jnp
from jax.experimental import pallas as pl
from jax.experimental.pallas import tpu as pltpu

N_NODES = 32
LANES = 128
ROWS_TOTAL = 8192
G = 2048
GS = 8
K = N_NODES * GS


def _body(xi_t_ref, s_nd_ref, x_ref, o_ref):
    xi_t = xi_t_ref[...]
    s_nd = s_nd_ref[...]
    for s in range(G // GS):
        x = x_ref[s * GS : (s + 1) * GS, :]
        d = jnp.reshape(x[:, None, :] - xi_t[None, :, :], (K, LANES))
        c = 1.0 / d
        c = jnp.minimum(c, 1e24)
        cb = c.astype(jnp.bfloat16)
        nd = jnp.dot(s_nd, cb, preferred_element_type=jnp.float32)
        out = nd[0:GS, :] / nd[GS : 2 * GS, :]
        o_ref[s * GS : (s + 1) * GS, :] = jnp.where(
            jnp.isfinite(out), out, x * x
        )


def kernel(x, xi, fi, wi):
    f32 = jnp.float32
    xq = x.reshape(ROWS_TOTAL, LANES)
    fw = fi * wi
    eye = jnp.eye(GS, dtype=f32)
    s_nd = jnp.concatenate(
        [jnp.kron(eye, fw[None, :]), jnp.kron(eye, wi[None, :])], axis=0
    ).astype(jnp.bfloat16)
    xi_t = jnp.broadcast_to(xi[:, None], (N_NODES, LANES))

    grid = (ROWS_TOTAL // G,)
    out = pl.pallas_call(
        _body,
        grid=grid,
        in_specs=[
            pl.BlockSpec((N_NODES, LANES), lambda i: (0, 0)),
            pl.BlockSpec((2 * GS, K), lambda i: (0, 0)),
            pl.BlockSpec((G, LANES), lambda i: (i, 0)),
        ],
        out_specs=pl.BlockSpec((G, LANES), lambda i: (i, 0)),
        out_shape=jax.ShapeDtypeStruct((ROWS_TOTAL, LANES), f32),
        compiler_params=pltpu.CompilerParams(
            dimension_semantics=("arbitrary",),
        ),
    )(xi_t, s_nd, xq)
    return out.reshape(-1)


# speedup vs baseline: 1.0791x; 1.0407x over previous
import jax
import jax.numpy as jnp
from jax.experimental import pallas as pl
from jax.experimental.pallas import tpu as pltpu

N_NODES = 32
LANES = 128
ROWS_TOTAL = 8192
G = 2048
GS = 8
K = N_NODES * GS


def _body(xi_t_ref, s_nd_ref, x_ref, o_ref):
    xi_t = xi_t_ref[...]
    s_nd = s_nd_ref[...]
    for s in range(G // GS):
        x = x_ref[s * GS : (s + 1) * GS, :]
        d = jnp.reshape(x[:, None, :] - xi_t[None, :, :], (K, LANES))
        c = 1.0 / d
        cb = jnp.minimum(c.astype(jnp.bfloat16), jnp.bfloat16(1.01e24))
        nd = jnp.dot(s_nd, cb, preferred_element_type=jnp.float32)
        out = nd[0:GS, :] / nd[GS : 2 * GS, :]
        o_ref[s * GS : (s + 1) * GS, :] = jnp.where(
            jnp.isfinite(out), out, x * x
        )


def kernel(x, xi, fi, wi):
    f32 = jnp.float32
    xq = x.reshape(ROWS_TOTAL, LANES)
    fw = fi * wi
    eye = jnp.eye(GS, dtype=f32)
    s_nd = jnp.concatenate(
        [jnp.kron(eye, fw[None, :]), jnp.kron(eye, wi[None, :])], axis=0
    ).astype(jnp.bfloat16)
    xi_t = jnp.broadcast_to(xi[:, None], (N_NODES, LANES))

    grid = (ROWS_TOTAL // G,)
    out = pl.pallas_call(
        _body,
        grid=grid,
        in_specs=[
            pl.BlockSpec((N_NODES, LANES), lambda i: (0, 0)),
            pl.BlockSpec((2 * GS, K), lambda i: (0, 0)),
            pl.BlockSpec((G, LANES), lambda i: (i, 0)),
        ],
        out_specs=pl.BlockSpec((G, LANES), lambda i: (i, 0)),
        out_shape=jax.ShapeDtypeStruct((ROWS_TOTAL, LANES), f32),
        compiler_params=pltpu.CompilerParams(
            dimension_semantics=("arbitrary",),
        ),
    )(xi_t, s_nd, xq)
    return out.reshape(-1)
